# baseline (speedup 1.0000x reference)
import jax
import jax.numpy as jnp
from jax.experimental import pallas as pl
from jax.experimental.pallas import tpu as pltpu

N_DAYS_YEAR = 365
N_STEPS = 48
_EPS = 1e-6
_R = 8
_BB = 32


def _emos_body(sid_ref, fp_ref, ft_ref, ct_ref, bt_ref, o_ref):
    base = pl.program_id(0) * _BB
    shape = fp_ref.shape[1:]
    lane = jax.lax.broadcasted_iota(jnp.int32, shape, 1)
    m4 = lane % 4
    sigma = (lane % 2) == 1

    for r in range(_BB):
        m = sid_ref[base + r]
        fp = fp_ref[r]
        ft = ft_ref[r]
        acc = jnp.where(sigma, jnp.log(fp + _EPS), fp) + bt_ref[m]
        for c in range(4):
            tk = ft * ct_ref[m, c]
            w = tk
            for i in range(4):
                if i != c:
                    w = w + jnp.roll(tk, c - i, axis=1)
            acc = jnp.where(m4 == c, acc + w, acc)
        o_ref[r] = jnp.where(sigma, jnp.exp(acc) - _EPS, acc)


@jax.jit
def kernel(day_of_year, step_idx, forecast_parameters, features, coefs, biases):
    NTM, NSM, S, IN_F, OUT_F, OUT_P = coefs.shape
    B = day_of_year.shape[0]
    NM = NTM * NSM
    C = OUT_F * OUT_P
    J = S * C
    L = J // _R

    time_span = -(-N_DAYS_YEAR // NTM)
    step_span = -(-N_STEPS // NSM)
    model_id = ((day_of_year // time_span) * NSM + (step_idx // step_span)).astype(jnp.int32)

    fp3 = forecast_parameters.reshape(B, _R, L)
    ft3 = features.reshape(B, _R, L)

    ct = coefs.reshape(NM, S, IN_F, C).transpose(0, 3, 1, 2).reshape(NM, C, _R, L)
    bt = biases.reshape(NM, _R, L)

    grid_spec = pltpu.PrefetchScalarGridSpec(
        num_scalar_prefetch=1,
        grid=(B // _BB,),
        in_specs=[
            pl.BlockSpec((_BB, _R, L), lambda i, s: (i, 0, 0)),
            pl.BlockSpec((_BB, _R, L), lambda i, s: (i, 0, 0)),
            pl.BlockSpec((NM, C, _R, L), lambda i, s: (0, 0, 0, 0)),
            pl.BlockSpec((NM, _R, L), lambda i, s: (0, 0, 0)),
        ],
        out_specs=pl.BlockSpec((_BB, _R, L), lambda i, s: (i, 0, 0)),
    )
    out = pl.pallas_call(
        _emos_body,
        grid_spec=grid_spec,
        out_shape=jax.ShapeDtypeStruct((B, _R, L), jnp.float32),
        compiler_params=pltpu.CompilerParams(
            dimension_semantics=("arbitrary",)),
    )(model_id, fp3, ft3, ct, bt)
    return out.reshape(B, S, OUT_F, OUT_P)
